# baseline (speedup 1.0000x reference)
import jax
import jax.numpy as jnp
from jax.experimental import pallas as pl
from jax.experimental.pallas import tpu as pltpu

_VOCAB = 100000
_D = 64
_BT = 256
_TILE_V = 16384


def _body(ids_ref, emb_hbm, w_ref, o_ref, h_scr, sem):
    @pl.when(pl.program_id(0) == 0)
    def _():
        for j in range(_BT):
            pltpu.make_async_copy(
                emb_hbm.at[ids_ref[j]], h_scr.at[j], sem).start()
        for j in range(_BT):
            pltpu.make_async_copy(
                emb_hbm.at[ids_ref[j]], h_scr.at[j], sem).wait()

    o_ref[...] = jnp.dot(h_scr[...], w_ref[...],
                         preferred_element_type=jnp.float32)


def kernel(x, emb, W):
    b, t = x.shape
    ids = x.reshape(_BT).astype(jnp.int32)
    nblk = pl.cdiv(_VOCAB, _TILE_V)
    grid_spec = pltpu.PrefetchScalarGridSpec(
        num_scalar_prefetch=1,
        grid=(nblk,),
        in_specs=[
            pl.BlockSpec(memory_space=pl.ANY),
            pl.BlockSpec((_D, _TILE_V), lambda i, ids_ref: (0, i)),
        ],
        out_specs=pl.BlockSpec((_BT, _TILE_V), lambda i, ids_ref: (0, i)),
        scratch_shapes=[
            pltpu.VMEM((_BT, _D), jnp.float32),
            pltpu.SemaphoreType.DMA,
        ],
    )
    logits = pl.pallas_call(
        _body,
        grid_spec=grid_spec,
        out_shape=jax.ShapeDtypeStruct((_BT, _VOCAB), jnp.float32),
        compiler_params=pltpu.CompilerParams(
            dimension_semantics=("arbitrary",)),
    )(ids, emb, W)
    return logits.reshape(b, t, _VOCAB)
